# baseline (speedup 1.0000x reference)
import functools

import jax
import jax.numpy as jnp
from jax import lax
from jax.experimental import pallas as pl
from jax.experimental.pallas import tpu as pltpu
from jax.experimental.pallas import tpu_sc as plsc

_NBUF = 4


def _sc_gather_pool(idx_flat, table, B, S, D, NC, NS):
    NW = NC * NS
    bpw = B // NW
    rows_pw = bpw * S
    NR = bpw // _NBUF
    S0 = 104
    S1 = S - S0
    NG = D // 16

    mesh = plsc.VectorSubcoreMesh(
        core_axis_name="c", subcore_axis_name="s",
        num_cores=NC, num_subcores=NS)

    @functools.partial(
        pl.kernel,
        out_type=(jax.ShapeDtypeStruct((B * S, D), jnp.float32),
                  jax.ShapeDtypeStruct((B * D,), jnp.float32)),
        mesh=mesh,
        compiler_params=pltpu.CompilerParams(use_tc_tiling_on_sc=False),
        scratch_types=(
            pltpu.VMEM((rows_pw,), jnp.int32),
            pltpu.VMEM((_NBUF, S, D), jnp.float32),
            pltpu.VMEM((bpw * D,), jnp.float32),
            pltpu.SemaphoreType.DMA,
            pltpu.SemaphoreType.DMA,
            pltpu.SemaphoreType.DMA,
            pltpu.SemaphoreType.DMA,
            pltpu.SemaphoreType.DMA,
            pltpu.SemaphoreType.DMA,
            pltpu.SemaphoreType.DMA,
            pltpu.SemaphoreType.DMA,
        ),
    )
    def body(idx_hbm, tab_hbm, tok_hbm, sum_hbm, idx_v, rows_v, acc_v,
             g0, g1, g2, g3, o0, o1, o2, o3):
        gsem = (g0, g1, g2, g3)
        osem = (o0, o1, o2, o3)
        wid = lax.axis_index("s") * NC + lax.axis_index("c")
        row0 = pl.multiple_of(wid * rows_pw, 8)
        pltpu.sync_copy(idx_hbm.at[pl.ds(row0, rows_pw)], idx_v)

        def issue_gather(k, ph):
            off = pl.multiple_of(k * S, 8)
            pltpu.async_copy(tab_hbm.at[idx_v.at[pl.ds(off, S0)]],
                             rows_v.at[ph, pl.ds(0, S0)], gsem[ph])
            off2 = pl.multiple_of(k * S + S0, 8)
            pltpu.async_copy(tab_hbm.at[idx_v.at[pl.ds(off2, S1)]],
                             rows_v.at[ph, pl.ds(S0, S1)], gsem[ph])

        def wait_gather(ph):
            pltpu.make_async_copy(tab_hbm.at[pl.ds(0, S)],
                                  rows_v.at[ph], gsem[ph]).wait()

        def issue_out(k, ph):
            pltpu.async_copy(rows_v.at[ph],
                             tok_hbm.at[pl.ds(row0 + k * S, S)], osem[ph])

        def wait_out(ph):
            pltpu.make_async_copy(rows_v.at[ph],
                                  tok_hbm.at[pl.ds(0, S)], osem[ph]).wait()

        def accumulate(k, ph):
            buf = rows_v.at[ph]
            zero = jnp.zeros((16,), jnp.float32)

            def sbody(s, c):
                return tuple(c[g] + buf[s, pl.ds(g * 16, 16)]
                             for g in range(NG))

            sums = lax.fori_loop(0, S, sbody, (zero,) * NG)
            base = k * D
            for g in range(NG):
                acc_v[pl.ds(base + g * 16, 16)] = sums[g]

        def step(k, ph, issue_next, wait_prev_out):
            wait_gather(ph)
            accumulate(k, ph)
            issue_out(k, ph)
            if issue_next:
                php = (ph + _NBUF - 1) % _NBUF
                if wait_prev_out:
                    wait_out(php)
                issue_gather(k + _NBUF - 1, php)

        for k in range(_NBUF - 1):
            issue_gather(k, k)

        step(0, 0, True, False)
        for ph in range(1, _NBUF):
            step(ph, ph, True, True)

        @pl.loop(1, NR - 1)
        def _(r):
            kb = r * _NBUF
            for ph in range(_NBUF):
                step(kb + ph, ph, True, True)

        kb = (NR - 1) * _NBUF
        step(kb, 0, True, True)
        for ph in range(1, _NBUF):
            step(kb + ph, ph, False, False)

        for ph in range(_NBUF):
            wait_out(ph)
        pltpu.sync_copy(
            acc_v, sum_hbm.at[pl.ds(pl.multiple_of(wid * bpw * D, 8),
                                    bpw * D)])

    return body(idx_flat, table)


def _proj_body(inv_s, s_ref, w_ref, b_ref, o_ref):
    x = s_ref[...] * inv_s
    o_ref[...] = lax.dot_general(
        x, w_ref[...], (((1,), (1,)), ((), ())),
        preferred_element_type=jnp.float32) + b_ref[...]


def kernel(input_ids, embed_table, W, b):
    B, S = input_ids.shape
    V, D = embed_table.shape
    idx = input_ids.reshape(B * S).astype(jnp.int32)
    try:
        info = plsc.get_sparse_core_info()
        NC, NS = info.num_cores, info.num_subcores
    except Exception:
        NC, NS = 2, 16

    tok_flat, sums = _sc_gather_pool(idx, embed_table, B, S, D, NC, NS)

    pooled = pl.pallas_call(
        functools.partial(_proj_body, 1.0 / S),
        out_shape=jax.ShapeDtypeStruct((B, D), jnp.float32),
    )(sums.reshape(B, D), W, b.reshape(1, D))

    return tok_flat.reshape(B, S, D), pooled


# speedup vs baseline: 1.2048x; 1.2048x over previous
import functools

import jax
import jax.numpy as jnp
from jax import lax
from jax.experimental import pallas as pl
from jax.experimental.pallas import tpu as pltpu
from jax.experimental.pallas import tpu_sc as plsc


def _sc_gather_pool(idx_flat, table_padded, B, S, D, NC, NS):
    NW = NC * NS
    bpw = B // NW
    rows_pw = bpw * S
    S0 = 104
    S1 = S - S0
    NG = D // 16
    DP = table_padded.shape[1]

    mesh = plsc.VectorSubcoreMesh(
        core_axis_name="c", subcore_axis_name="s",
        num_cores=NC, num_subcores=NS)

    @functools.partial(
        pl.kernel,
        out_type=(jax.ShapeDtypeStruct((B * S, D), jnp.float32),
                  jax.ShapeDtypeStruct((B, D), jnp.float32)),
        mesh=mesh,
        compiler_params=pltpu.CompilerParams(use_tc_tiling_on_sc=True),
        scratch_types=(
            pltpu.VMEM((rows_pw,), jnp.int32),
            pltpu.VMEM((2, S, DP), jnp.float32),
            pltpu.VMEM((2, S0, D), jnp.float32),
            pltpu.VMEM((bpw, D), jnp.float32),
            pltpu.SemaphoreType.DMA,
            pltpu.SemaphoreType.DMA,
            pltpu.SemaphoreType.DMA,
            pltpu.SemaphoreType.DMA,
        ),
    )
    def body(idx_hbm, tab_hbm, tok_hbm, sum_hbm, idx_v, rows_v, stg_v,
             acc_v, g0, g1, o0, o1):
        gsem = (g0, g1)
        osem = (o0, o1)
        wid = lax.axis_index("s") * NC + lax.axis_index("c")
        row0 = pl.multiple_of(wid * rows_pw, 8)
        pltpu.sync_copy(idx_hbm.at[pl.ds(row0, rows_pw)], idx_v)

        def issue_gather(k, ph):
            off = pl.multiple_of(k * S, 8)
            pltpu.async_copy(tab_hbm.at[idx_v.at[pl.ds(off, S0)]],
                             rows_v.at[ph, pl.ds(0, S0)], gsem[ph])
            off2 = pl.multiple_of(k * S + S0, 8)
            pltpu.async_copy(tab_hbm.at[idx_v.at[pl.ds(off2, S1)]],
                             rows_v.at[ph, pl.ds(S0, S1)], gsem[ph])

        def wait_gather(ph):
            pltpu.make_async_copy(tab_hbm.at[pl.ds(0, S)],
                                  rows_v.at[ph], gsem[ph]).wait()

        def repack_half(ph, q, base, n, carry):
            buf = rows_v.at[ph]
            stg = stg_v.at[q]

            def sbody(s, c):
                out = []
                for g in range(NG):
                    v = buf[base + s, pl.ds(g * 16, 16)]
                    stg[s, pl.ds(g * 16, 16)] = v
                    out.append(c[g] + v)
                return tuple(out)

            return lax.fori_loop(0, n, sbody, carry)

        def issue_out(k, q, base, n):
            pltpu.async_copy(
                stg_v.at[q, pl.ds(0, n)],
                tok_hbm.at[pl.ds(row0 + k * S + base, n)], osem[q])

        def wait_out(q, n):
            pltpu.make_async_copy(stg_v.at[q, pl.ds(0, n)],
                                  tok_hbm.at[pl.ds(0, n)], osem[q]).wait()

        def do_batch(k, ph, first_k, last_k):
            wait_gather(ph)
            if not last_k:
                issue_gather(k + 1, 1 - ph)
            zero = jnp.zeros((16,), jnp.float32)
            if not first_k:
                wait_out(0, S0)
            c = repack_half(ph, 0, 0, S0, (zero,) * NG)
            issue_out(k, 0, 0, S0)
            if not first_k:
                wait_out(1, S1)
            c = repack_half(ph, 1, S0, S1, c)
            issue_out(k, 1, S0, S1)
            for g in range(NG):
                acc_v[k, pl.ds(g * 16, 16)] = c[g]

        issue_gather(0, 0)
        do_batch(0, 0, True, False)

        @pl.loop(0, (bpw - 2) // 2)
        def _(r):
            k = 1 + 2 * r
            do_batch(k, 1, False, False)
            do_batch(k + 1, 0, False, False)

        do_batch(bpw - 1, 1, False, True)

        wait_out(0, S0)
        wait_out(1, S1)
        pltpu.sync_copy(
            acc_v, sum_hbm.at[pl.ds(pl.multiple_of(wid * bpw, 8), bpw)])

    return body(idx_flat, table_padded)


def _proj_body(inv_s, s_ref, w_ref, b_ref, o_ref):
    x = s_ref[...] * inv_s
    o_ref[...] = lax.dot_general(
        x, w_ref[...], (((1,), (1,)), ((), ())),
        preferred_element_type=jnp.float32) + b_ref[...]


def kernel(input_ids, embed_table, W, b):
    B, S = input_ids.shape
    V, D = embed_table.shape
    idx = input_ids.reshape(B * S).astype(jnp.int32)
    try:
        info = plsc.get_sparse_core_info()
        NC, NS = info.num_cores, info.num_subcores
    except Exception:
        NC, NS = 2, 16

    pad = (-D) % 128
    table_padded = jnp.pad(embed_table, ((0, 0), (0, pad)))
    tok_flat, sums = _sc_gather_pool(idx, table_padded, B, S, D, NC, NS)

    pooled = pl.pallas_call(
        functools.partial(_proj_body, 1.0 / S),
        out_shape=jax.ShapeDtypeStruct((B, D), jnp.float32),
    )(sums, W, b.reshape(1, D))

    return tok_flat.reshape(B, S, D), pooled


# speedup vs baseline: 1.2265x; 1.0180x over previous
import functools

import jax
import jax.numpy as jnp
from jax import lax
from jax.experimental import pallas as pl
from jax.experimental.pallas import tpu as pltpu
from jax.experimental.pallas import tpu_sc as plsc


def _sc_gather_pool(idx_flat, table_padded, B, S, D, NC, NS):
    NW = NC * NS
    bpw = B // NW
    rows_pw = bpw * S
    S0 = 104
    S1 = S - S0
    NG = D // 16
    DP = table_padded.shape[1]

    mesh = plsc.VectorSubcoreMesh(
        core_axis_name="c", subcore_axis_name="s",
        num_cores=NC, num_subcores=NS)

    @functools.partial(
        pl.kernel,
        out_type=(jax.ShapeDtypeStruct((B * S, D), jnp.float32),
                  jax.ShapeDtypeStruct((B, D), jnp.float32)),
        mesh=mesh,
        compiler_params=pltpu.CompilerParams(use_tc_tiling_on_sc=True),
        scratch_types=(
            pltpu.VMEM((rows_pw,), jnp.int32),
            pltpu.VMEM((2, S, DP), jnp.float32),
            pltpu.VMEM((2, S0, D), jnp.float32),
            pltpu.VMEM((bpw, D), jnp.float32),
            pltpu.SemaphoreType.DMA,
            pltpu.SemaphoreType.DMA,
            pltpu.SemaphoreType.DMA,
            pltpu.SemaphoreType.DMA,
        ),
    )
    def body(idx_hbm, tab_hbm, tok_hbm, sum_hbm, idx_v, rows_v, stg_v,
             acc_v, g0, g1, o0, o1):
        gsem = (g0, g1)
        osem = (o0, o1)
        wid = lax.axis_index("s") * NC + lax.axis_index("c")
        row0 = pl.multiple_of(wid * rows_pw, 8)
        pltpu.sync_copy(idx_hbm.at[pl.ds(row0, rows_pw)], idx_v)

        def issue_gather(k, ph):
            off = pl.multiple_of(k * S, 8)
            pltpu.async_copy(tab_hbm.at[idx_v.at[pl.ds(off, S0)]],
                             rows_v.at[ph, pl.ds(0, S0)], gsem[ph])
            off2 = pl.multiple_of(k * S + S0, 8)
            pltpu.async_copy(tab_hbm.at[idx_v.at[pl.ds(off2, S1)]],
                             rows_v.at[ph, pl.ds(S0, S1)], gsem[ph])

        def wait_gather(ph):
            pltpu.make_async_copy(tab_hbm.at[pl.ds(0, S)],
                                  rows_v.at[ph], gsem[ph]).wait()

        def repack_half(ph, q, base, n, carry):
            buf = rows_v.at[ph]
            stg = stg_v.at[q]

            def sbody(s, c):
                out = []
                for g in range(NG):
                    v = buf[base + s, pl.ds(g * 16, 16)]
                    stg[s, pl.ds(g * 16, 16)] = v
                    out.append(c[g] + v)
                return tuple(out)

            return lax.fori_loop(0, n, sbody, carry, unroll=8)

        def issue_out(k, q, base, n):
            pltpu.async_copy(
                stg_v.at[q, pl.ds(0, n)],
                tok_hbm.at[pl.ds(row0 + k * S + base, n)], osem[q])

        def wait_out(q, n):
            pltpu.make_async_copy(stg_v.at[q, pl.ds(0, n)],
                                  tok_hbm.at[pl.ds(0, n)], osem[q]).wait()

        def do_batch(k, ph, first_k, last_k):
            wait_gather(ph)
            if not last_k:
                issue_gather(k + 1, 1 - ph)
            zero = jnp.zeros((16,), jnp.float32)
            if not first_k:
                wait_out(0, S0)
            c = repack_half(ph, 0, 0, S0, (zero,) * NG)
            issue_out(k, 0, 0, S0)
            if not first_k:
                wait_out(1, S1)
            c = repack_half(ph, 1, S0, S1, c)
            issue_out(k, 1, S0, S1)
            for g in range(NG):
                acc_v[k, pl.ds(g * 16, 16)] = c[g]

        issue_gather(0, 0)
        do_batch(0, 0, True, False)

        @pl.loop(0, (bpw - 2) // 2)
        def _(r):
            k = 1 + 2 * r
            do_batch(k, 1, False, False)
            do_batch(k + 1, 0, False, False)

        do_batch(bpw - 1, 1, False, True)

        wait_out(0, S0)
        wait_out(1, S1)
        pltpu.sync_copy(
            acc_v, sum_hbm.at[pl.ds(pl.multiple_of(wid * bpw, 8), bpw)])

    return body(idx_flat, table_padded)


def _proj_body(inv_s, s_ref, w_ref, b_ref, o_ref):
    x = s_ref[...] * inv_s
    o_ref[...] = lax.dot_general(
        x, w_ref[...], (((1,), (1,)), ((), ())),
        preferred_element_type=jnp.float32) + b_ref[...]


def _prep_body(D, tabT_ref, o_ref):
    x = tabT_ref[...]
    eye = jnp.eye(D, dtype=jnp.float32)
    o_ref[:, :D] = lax.dot_general(
        x, eye, (((0,), (0,)), ((), ())),
        preferred_element_type=jnp.float32)


def _prep_table(embed_table, V, D):
    VB = 2048
    grid = (V + VB - 1) // VB
    return pl.pallas_call(
        functools.partial(_prep_body, D),
        grid=(grid,),
        in_specs=[pl.BlockSpec((D, VB), lambda j: (0, j))],
        out_specs=pl.BlockSpec((VB, 128), lambda j: (j, 0)),
        out_shape=jax.ShapeDtypeStruct((V, 128), jnp.float32),
    )(embed_table.T)


def kernel(input_ids, embed_table, W, b):
    B, S = input_ids.shape
    V, D = embed_table.shape
    idx = input_ids.reshape(B * S).astype(jnp.int32)
    try:
        info = plsc.get_sparse_core_info()
        NC, NS = info.num_cores, info.num_subcores
    except Exception:
        NC, NS = 2, 16

    table_padded = _prep_table(embed_table, V, D)
    tok_flat, sums = _sc_gather_pool(idx, table_padded, B, S, D, NC, NS)

    pooled = pl.pallas_call(
        functools.partial(_proj_body, 1.0 / S),
        out_shape=jax.ShapeDtypeStruct((B, D), jnp.float32),
    )(sums, W, b.reshape(1, D))

    return tok_flat.reshape(B, S, D), pooled


# speedup vs baseline: 1.2555x; 1.0237x over previous
import functools

import jax
import jax.numpy as jnp
from jax import lax
from jax.experimental import pallas as pl
from jax.experimental.pallas import tpu as pltpu
from jax.experimental.pallas import tpu_sc as plsc


def _sc_gather_pool(idx_flat, table_padded, B, S, D, NC, NS):
    NW = NC * NS
    bpw = B // NW
    rows_pw = bpw * S
    S0 = 104
    S1 = S - S0
    NG = D // 16
    DP = table_padded.shape[1]

    mesh = plsc.VectorSubcoreMesh(
        core_axis_name="c", subcore_axis_name="s",
        num_cores=NC, num_subcores=NS)

    @functools.partial(
        pl.kernel,
        out_type=(jax.ShapeDtypeStruct((B * S, D), jnp.float32),
                  jax.ShapeDtypeStruct((B, D), jnp.float32)),
        mesh=mesh,
        compiler_params=pltpu.CompilerParams(use_tc_tiling_on_sc=True),
        scratch_types=(
            pltpu.VMEM((rows_pw,), jnp.int32),
            pltpu.VMEM((2, S, DP), jnp.float32),
            pltpu.VMEM((2, S0, D), jnp.float32),
            pltpu.VMEM((bpw, D), jnp.float32),
            pltpu.SemaphoreType.DMA,
            pltpu.SemaphoreType.DMA,
            pltpu.SemaphoreType.DMA,
            pltpu.SemaphoreType.DMA,
        ),
    )
    def body(idx_hbm, tab_hbm, tok_hbm, sum_hbm, idx_v, rows_v, stg_v,
             acc_v, g0, g1, o0, o1):
        gsem = (g0, g1)
        osem = (o0, o1)
        wid = lax.axis_index("s") * NC + lax.axis_index("c")
        row0 = pl.multiple_of(wid * rows_pw, 8)
        pltpu.sync_copy(idx_hbm.at[pl.ds(row0, rows_pw)], idx_v)

        def issue_gather(k, ph):
            off = pl.multiple_of(k * S, 8)
            pltpu.async_copy(tab_hbm.at[idx_v.at[pl.ds(off, S0)]],
                             rows_v.at[ph, pl.ds(0, S0)], gsem[ph])
            off2 = pl.multiple_of(k * S + S0, 8)
            pltpu.async_copy(tab_hbm.at[idx_v.at[pl.ds(off2, S1)]],
                             rows_v.at[ph, pl.ds(S0, S1)], gsem[ph])

        def wait_gather(ph):
            pltpu.make_async_copy(tab_hbm.at[pl.ds(0, S)],
                                  rows_v.at[ph], gsem[ph]).wait()

        def repack_half(ph, q, base, n, carry):
            buf = rows_v.at[ph]
            stg = stg_v.at[q]

            def sbody(s, c):
                out = []
                for g in range(NG):
                    v = buf[base + s, pl.ds(g * 16, 16)]
                    stg[s, pl.ds(g * 16, 16)] = v
                    out.append(c[g] + v)
                return tuple(out)

            return lax.fori_loop(0, n, sbody, carry, unroll=8)

        def issue_out(k, q, base, n):
            pltpu.async_copy(
                stg_v.at[q, pl.ds(0, n)],
                tok_hbm.at[pl.ds(row0 + k * S + base, n)], osem[q])

        def wait_out(q, n):
            pltpu.make_async_copy(stg_v.at[q, pl.ds(0, n)],
                                  tok_hbm.at[pl.ds(0, n)], osem[q]).wait()

        def do_batch(k, ph, first_k, last_k):
            wait_gather(ph)
            if not last_k:
                issue_gather(k + 1, 1 - ph)
            zero = jnp.zeros((16,), jnp.float32)
            if not first_k:
                wait_out(0, S0)
            c = repack_half(ph, 0, 0, S0, (zero,) * NG)
            issue_out(k, 0, 0, S0)
            if not first_k:
                wait_out(1, S1)
            c = repack_half(ph, 1, S0, S1, c)
            issue_out(k, 1, S0, S1)
            for g in range(NG):
                acc_v[k, pl.ds(g * 16, 16)] = c[g]

        issue_gather(0, 0)
        do_batch(0, 0, True, False)

        @pl.loop(0, (bpw - 2) // 2)
        def _(r):
            k = 1 + 2 * r
            do_batch(k, 1, False, False)
            do_batch(k + 1, 0, False, False)

        do_batch(bpw - 1, 1, False, True)

        wait_out(0, S0)
        wait_out(1, S1)
        pltpu.sync_copy(
            acc_v, sum_hbm.at[pl.ds(pl.multiple_of(wid * bpw, 8), bpw)])

    return body(idx_flat, table_padded)


def _proj_body(inv_s, s_ref, w_ref, b_ref, o_ref):
    x = s_ref[...] * inv_s
    o_ref[...] = lax.dot_general(
        x, w_ref[...], (((1,), (1,)), ((), ())),
        preferred_element_type=jnp.float32) + b_ref[...]


def _prep_body(D, tabT_ref, o_ref):
    o_ref[:, :D] = jnp.transpose(tabT_ref[...], (1, 0))


def _prep_table(embed_table, V, D):
    VB = 2048
    grid = (V + VB - 1) // VB
    return pl.pallas_call(
        functools.partial(_prep_body, D),
        grid=(grid,),
        in_specs=[pl.BlockSpec((D, VB), lambda j: (0, j))],
        out_specs=pl.BlockSpec((VB, 128), lambda j: (j, 0)),
        out_shape=jax.ShapeDtypeStruct((V, 128), jnp.float32),
    )(embed_table.T)


def kernel(input_ids, embed_table, W, b):
    B, S = input_ids.shape
    V, D = embed_table.shape
    idx = input_ids.reshape(B * S).astype(jnp.int32)
    try:
        info = plsc.get_sparse_core_info()
        NC, NS = info.num_cores, info.num_subcores
    except Exception:
        NC, NS = 2, 16

    table_padded = _prep_table(embed_table, V, D)
    tok_flat, sums = _sc_gather_pool(idx, table_padded, B, S, D, NC, NS)

    pooled = pl.pallas_call(
        functools.partial(_proj_body, 1.0 / S),
        out_shape=jax.ShapeDtypeStruct((B, D), jnp.float32),
    )(sums, W, b.reshape(1, D))

    return tok_flat.reshape(B, S, D), pooled


# speedup vs baseline: 1.5611x; 1.2434x over previous
import functools

import jax
import jax.numpy as jnp
from jax import lax
from jax.experimental import pallas as pl
from jax.experimental.pallas import tpu as pltpu
from jax.experimental.pallas import tpu_sc as plsc


def _sc_gather_pool(idx_flat, table_padded, B, S, D, NC, NS):
    NW = NC * NS
    bpw = B // NW
    rows_pw = bpw * S
    S0 = 104
    S1 = S - S0
    NG = D // 16
    DP = table_padded.shape[1]

    mesh = plsc.VectorSubcoreMesh(
        core_axis_name="c", subcore_axis_name="s",
        num_cores=NC, num_subcores=NS)

    @functools.partial(
        pl.kernel,
        out_type=(jax.ShapeDtypeStruct((B * S, D), jnp.float32),
                  jax.ShapeDtypeStruct((B, D), jnp.float32)),
        mesh=mesh,
        compiler_params=pltpu.CompilerParams(
            use_tc_tiling_on_sc=True, disable_semaphore_checks=True),
        scratch_types=(
            pltpu.VMEM((rows_pw,), jnp.int32),
            pltpu.VMEM((2, S, DP), jnp.float32),
            pltpu.VMEM((2, S0, D), jnp.float32),
            pltpu.VMEM((bpw, D), jnp.float32),
            pltpu.SemaphoreType.DMA,
            pltpu.SemaphoreType.DMA,
            pltpu.SemaphoreType.DMA,
            pltpu.SemaphoreType.DMA,
        ),
    )
    def body(idx_hbm, tab_hbm, tok_hbm, sum_hbm, idx_v, rows_v, stg_v,
             acc_v, g0, g1, o0, o1):
        gsem = (g0, g1)
        osem = (o0, o1)
        wid = lax.axis_index("s") * NC + lax.axis_index("c")
        row0 = pl.multiple_of(wid * rows_pw, 8)
        pltpu.sync_copy(idx_hbm.at[pl.ds(row0, rows_pw)], idx_v)

        def issue_gather(k, ph):
            off = pl.multiple_of(k * S, 8)
            pltpu.async_copy(tab_hbm.at[idx_v.at[pl.ds(off, S0)]],
                             rows_v.at[ph, pl.ds(0, S0)], gsem[ph])
            off2 = pl.multiple_of(k * S + S0, 8)
            pltpu.async_copy(tab_hbm.at[idx_v.at[pl.ds(off2, S1)]],
                             rows_v.at[ph, pl.ds(S0, S1)], gsem[ph])

        def wait_gather(ph):
            pltpu.make_async_copy(tab_hbm.at[pl.ds(0, S)],
                                  rows_v.at[ph], gsem[ph]).wait()

        def repack_half(ph, q, base, n, carry):
            buf = rows_v.at[ph]
            stg = stg_v.at[q]

            def sbody(s, c):
                out = []
                for g in range(NG):
                    v = buf[base + s, pl.ds(g * 16, 16)]
                    stg[s, pl.ds(g * 16, 16)] = v
                    out.append(c[g] + v)
                return tuple(out)

            return lax.fori_loop(0, n, sbody, carry, unroll=2)

        def issue_out(k, q, base, n):
            pltpu.async_copy(
                stg_v.at[q, pl.ds(0, n)],
                tok_hbm.at[pl.ds(row0 + k * S + base, n)], osem[q])

        def wait_out(q, n):
            pltpu.make_async_copy(stg_v.at[q, pl.ds(0, n)],
                                  tok_hbm.at[pl.ds(0, n)], osem[q]).wait()

        def do_batch(k, ph, first_k, last_k):
            wait_gather(ph)
            if not last_k:
                issue_gather(k + 1, 1 - ph)
            zero = jnp.zeros((16,), jnp.float32)
            if not first_k:
                wait_out(0, S0)
            c = repack_half(ph, 0, 0, S0, (zero,) * NG)
            issue_out(k, 0, 0, S0)
            if not first_k:
                wait_out(1, S1)
            c = repack_half(ph, 1, S0, S1, c)
            issue_out(k, 1, S0, S1)
            for g in range(NG):
                acc_v[k, pl.ds(g * 16, 16)] = c[g]

        issue_gather(0, 0)
        do_batch(0, 0, True, False)

        @pl.loop(0, (bpw - 2) // 2)
        def _(r):
            k = 1 + 2 * r
            do_batch(k, 1, False, False)
            do_batch(k + 1, 0, False, False)

        do_batch(bpw - 1, 1, False, True)

        wait_out(0, S0)
        wait_out(1, S1)
        pltpu.sync_copy(
            acc_v, sum_hbm.at[pl.ds(pl.multiple_of(wid * bpw, 8), bpw)])

    return body(idx_flat, table_padded)


def _proj_body(inv_s, s_ref, w_ref, b_ref, o_ref):
    x = s_ref[...] * inv_s
    o_ref[...] = lax.dot_general(
        x, w_ref[...], (((1,), (1,)), ((), ())),
        preferred_element_type=jnp.float32) + b_ref[...]


def _prep_body(D, tabT_ref, o_ref):
    o_ref[:, :D] = jnp.transpose(tabT_ref[...], (1, 0))


def _prep_table(embed_table, V, D):
    VB = 8192
    grid = (V + VB - 1) // VB
    return pl.pallas_call(
        functools.partial(_prep_body, D),
        grid=(grid,),
        in_specs=[pl.BlockSpec((D, VB), lambda j: (0, j))],
        out_specs=pl.BlockSpec((VB, 128), lambda j: (j, 0)),
        out_shape=jax.ShapeDtypeStruct((V, 128), jnp.float32),
    )(embed_table.T)


def kernel(input_ids, embed_table, W, b):
    B, S = input_ids.shape
    V, D = embed_table.shape
    idx = input_ids.reshape(B * S).astype(jnp.int32)
    try:
        info = plsc.get_sparse_core_info()
        NC, NS = info.num_cores, info.num_subcores
    except Exception:
        NC, NS = 2, 16

    table_padded = _prep_table(embed_table, V, D)
    tok_flat, sums = _sc_gather_pool(idx, table_padded, B, S, D, NC, NS)

    pooled = pl.pallas_call(
        functools.partial(_proj_body, 1.0 / S),
        out_shape=jax.ShapeDtypeStruct((B, D), jnp.float32),
    )(sums, W, b.reshape(1, D))

    return tok_flat.reshape(B, S, D), pooled
